# baseline (speedup 1.0000x reference)
import functools

import jax
import jax.numpy as jnp
from jax import lax
from jax.experimental import pallas as pl
from jax.experimental.pallas import tpu as pltpu
from jax.experimental.pallas import tpu_sc as plsc

TOPK = 2
TILE_M = 256
TILE_SHIFT = 8
NUM_TILES = 24
NSLOT = NUM_TILES * TILE_M
SCAN_CHUNK = 512


def _router_body(x_ref, gw_ref, pos_ref, rw_ref, ex_ref, used_ref):
    x = x_ref[...]
    gw = gw_ref[...]
    E = gw.shape[0]
    T = x.shape[0]
    logits = lax.dot_general(gw, x, (((1,), (1,)), ((), ())),
                             preferred_element_type=jnp.float32)
    row = lax.broadcasted_iota(jnp.int32, (E, T), 0)
    v0 = jnp.max(logits, axis=0, keepdims=True)
    a0 = jnp.min(jnp.where(logits == v0, row, E), axis=0, keepdims=True)
    masked = jnp.where(row == a0, -jnp.inf, logits)
    v1 = jnp.max(masked, axis=0, keepdims=True)
    a1 = jnp.min(jnp.where(masked == v1, row, E), axis=0, keepdims=True)
    d = jnp.exp(v1 - v0)
    w0 = 1.0 / (1.0 + d)
    w1 = d / (1.0 + d)

    oh0 = (row == a0).astype(jnp.float32)
    oh1 = (row == a1).astype(jnp.float32)

    C = SCAN_CHUNK
    ci = lax.broadcasted_iota(jnp.int32, (C, C), 0)
    cj = lax.broadcasted_iota(jnp.int32, (C, C), 1)
    upper_incl = (ci <= cj).astype(jnp.float32)
    carry = jnp.zeros((E, 1), jnp.float32)
    ranks = []
    for oh in (oh0, oh1):
        chunks = []
        for c in range(T // C):
            ohc = oh[:, c * C:(c + 1) * C]
            run = lax.dot_general(ohc, upper_incl, (((1,), (0,)), ((), ())),
                                  preferred_element_type=jnp.float32) + carry
            chunks.append(jnp.sum(run * ohc, axis=0, keepdims=True))
            carry = run[:, C - 1:C]
        ranks.append(jnp.concatenate(chunks, axis=1) - 1.0)
    counts = carry

    counts_i = counts.astype(jnp.int32)
    tiles = lax.shift_right_logical(counts_i + (TILE_M - 1), TILE_SHIFT)
    tiles_f = tiles.astype(jnp.float32)
    ei = lax.broadcasted_iota(jnp.int32, (E, E), 0)
    ej = lax.broadcasted_iota(jnp.int32, (E, E), 1)
    strict_lower = (ej < ei).astype(jnp.float32)
    tbase = lax.dot_general(strict_lower, tiles_f, (((1,), (0,)), ((), ())),
                            preferred_element_type=jnp.float32)
    pbase = tbase * float(TILE_M)

    pos0 = jnp.sum(oh0 * pbase, axis=0, keepdims=True) + ranks[0]
    pos1 = jnp.sum(oh1 * pbase, axis=0, keepdims=True) + ranks[1]
    pos_ref[0:1, :] = pos0.astype(jnp.int32)
    pos_ref[1:2, :] = pos1.astype(jnp.int32)
    rw_ref[0:1, :] = w0
    rw_ref[1:2, :] = w1

    g = lax.broadcasted_iota(jnp.int32, (1, 32), 1)
    owner = jnp.sum((tbase <= g.astype(jnp.float32)).astype(jnp.float32),
                    axis=0, keepdims=True).astype(jnp.int32) - 1
    total = jnp.sum(tiles_f).astype(jnp.int32)
    ex_ref[...] = jnp.where(g < total, owner, -1)
    used_ref[...] = jnp.zeros((1, 16), jnp.int32) + total * TILE_M


def _router(x, gate_w):
    T, D = x.shape
    return pl.pallas_call(
        _router_body,
        out_shape=(
            jax.ShapeDtypeStruct((2, T), jnp.int32),
            jax.ShapeDtypeStruct((2, T), jnp.float32),
            jax.ShapeDtypeStruct((1, 32), jnp.int32),
            jax.ShapeDtypeStruct((1, 16), jnp.int32),
        ),
    )(x, gate_w)


HALF = NSLOT // 2


def _pipelined_gather(x_hbm, dst_hbm, idx_ref, dst_base, idx_base, used,
                      glob_base, ch, rows0, rows1, sg0, sg1, sw0, sw1):
    def copy_in(c, buf, sem):
        idx = idx_ref.at[pl.ds(idx_base + c * ch, ch)]
        return pltpu.make_async_copy(x_hbm.at[idx], buf, sem)

    def copy_out(c, buf, sem):
        return pltpu.make_async_copy(
            buf, dst_hbm.at[pl.ds(dst_base + c * ch, ch)], sem)

    live = [glob_base + c * ch < used for c in range(2)]
    pl.when(live[0])(lambda: copy_in(0, rows0, sg0).start())
    pl.when(live[1])(lambda: copy_in(1, rows1, sg1).start())

    def drain0():
        copy_in(0, rows0, sg0).wait()
        copy_out(0, rows0, sw0).start()
    pl.when(live[0])(drain0)

    def drain1():
        copy_in(1, rows1, sg1).wait()
        copy_out(1, rows1, sw1).start()
    pl.when(live[1])(drain1)
    pl.when(live[0])(lambda: copy_out(0, rows0, sw0).wait())
    pl.when(live[1])(lambda: copy_out(1, rows1, sw1).wait())


def _dispatch_a_body(x_hbm, pos_hbm, used_hbm,
                     xs_hbm, gidx_hbm,
                     pos_v, used_v, gidx_v, rows0, rows1,
                     sg0, sg1, sw0, sw1):
    T, D = x_hbm.shape
    nw = 32
    spt = NSLOT // nw
    spa = HALF // nw
    ch = spa // 2
    wid = lax.axis_index("s") * 2 + lax.axis_index("c")
    base = wid * spt
    basea = wid * spa

    with jax.named_scope("disp_meta"):
        pltpu.sync_copy(pos_hbm, pos_v)
        pltpu.sync_copy(used_hbm, used_v)

        def init(i, c):
            off = base + i * 16
            gidx_v[pl.ds(off, 16)] = jnp.bitwise_and(
                lax.iota(jnp.int32, 16) + off, T - 1)
            offa = basea + i * 16
            gidx_v[pl.ds(offa, 16)] = jnp.bitwise_and(
                lax.iota(jnp.int32, 16) + offa, T - 1)
            return c
        lax.fori_loop(0, spt // 16, init, 0)

        def scat(j, c):
            tid = lax.iota(jnp.int32, 16) + j * 16
            for k in range(TOPK):
                p = pos_v[k, pl.ds(j * 16, 16)]
                plsc.store_scatter(gidx_v, [p], tid)
            return c
        lax.fori_loop(0, T // 16, scat, 0)

        pltpu.sync_copy(gidx_v.at[pl.ds(base, spt)], gidx_hbm.at[pl.ds(base, spt)])

    used = used_v[0, pl.ds(0, 16)][0]
    with jax.named_scope("disp_gather"):
        _pipelined_gather(x_hbm, xs_hbm, gidx_v, basea, basea, used,
                          basea, ch, rows0, rows1, sg0, sg1, sw0, sw1)


def _dispatch_a(x, pos, used):
    T, D = x.shape
    mesh = plsc.VectorSubcoreMesh(core_axis_name="c", subcore_axis_name="s")
    ch = HALF // 32 // 2
    return pl.kernel(
        _dispatch_a_body,
        out_type=(
            jax.ShapeDtypeStruct((HALF, D), jnp.float32),
            jax.ShapeDtypeStruct((NSLOT,), jnp.int32),
        ),
        mesh=mesh,
        compiler_params=pltpu.CompilerParams(needs_layout_passes=False),
        scratch_types=[
            pltpu.VMEM((2, T), jnp.int32),
            pltpu.VMEM((1, 16), jnp.int32),
            pltpu.VMEM((NSLOT,), jnp.int32),
            pltpu.VMEM((ch, D), jnp.float32),
            pltpu.VMEM((ch, D), jnp.float32),
            pltpu.SemaphoreType.DMA,
            pltpu.SemaphoreType.DMA,
            pltpu.SemaphoreType.DMA,
            pltpu.SemaphoreType.DMA,
        ],
    )(x, pos, used)


def _dispatch_b_body(x_hbm, gidx_hbm, used_hbm, xs_hbm,
                     gixb_v, used_v, rows0, rows1, sg0, sg1, sw0, sw1):
    T, D = x_hbm.shape
    nw = 32
    spb = HALF // nw
    ch = spb // 2
    wid = lax.axis_index("s") * 2 + lax.axis_index("c")
    base = wid * spb
    gbase = HALF + base

    pltpu.sync_copy(gidx_hbm.at[pl.ds(gbase, spb)], gixb_v)
    pltpu.sync_copy(used_hbm, used_v)
    used = used_v[0, pl.ds(0, 16)][0]
    _pipelined_gather(x_hbm, xs_hbm, gixb_v, base, 0, used,
                      gbase, ch, rows0, rows1, sg0, sg1, sw0, sw1)


def _dispatch_b(x, gidx, used):
    T, D = x.shape
    mesh = plsc.VectorSubcoreMesh(core_axis_name="c", subcore_axis_name="s")
    ch = HALF // 32 // 2
    return pl.kernel(
        _dispatch_b_body,
        out_type=jax.ShapeDtypeStruct((HALF, D), jnp.float32),
        mesh=mesh,
        compiler_params=pltpu.CompilerParams(needs_layout_passes=False),
        scratch_types=[
            pltpu.VMEM((HALF // 32,), jnp.int32),
            pltpu.VMEM((1, 16), jnp.int32),
            pltpu.VMEM((ch, D), jnp.float32),
            pltpu.VMEM((ch, D), jnp.float32),
            pltpu.SemaphoreType.DMA,
            pltpu.SemaphoreType.DMA,
            pltpu.SemaphoreType.DMA,
            pltpu.SemaphoreType.DMA,
        ],
    )(x, gidx, used)


def _ffn_body(ex_ref, x_ref, w1_ref, w3_ref, w2_ref, y_ref, lo=0):
    j = pl.program_id(1)

    @pl.when(ex_ref[pl.program_id(0) + lo] >= 0)
    def _():
        x = x_ref[...]
        g = lax.dot_general(x, w1_ref[0], (((1,), (1,)), ((), ())),
                            preferred_element_type=jnp.float32)
        u = lax.dot_general(x, w3_ref[0], (((1,), (1,)), ((), ())),
                            preferred_element_type=jnp.float32)
        h = g * jax.nn.sigmoid(g) * u
        yj = lax.dot_general(h, w2_ref[0], (((1,), (1,)), ((), ())),
                             preferred_element_type=jnp.float32)

        @pl.when(j == 0)
        def _():
            y_ref[...] = yj

        @pl.when(j > 0)
        def _():
            y_ref[...] = y_ref[...] + yj


def _ffn_half(ex, xs_half, w1, w3, w2, lo, y_prev=None):
    E, F, D = w1.shape
    nt = NUM_TILES // 2

    def wsel_f(g, j, ex_s):
        return (jnp.maximum(ex_s[g + lo], 0), j, 0)

    def wsel_d(g, j, ex_s):
        return (jnp.maximum(ex_s[g + lo], 0), 0, j)

    in_specs = [
        pl.BlockSpec((TILE_M, D), lambda g, j, ex_s: (g, 0)),
        pl.BlockSpec((1, F // 2, D), wsel_f),
        pl.BlockSpec((1, F // 2, D), wsel_f),
        pl.BlockSpec((1, D, F // 2), wsel_d),
    ]
    args = [ex, xs_half, w1, w3, w2]
    kwargs = {}
    if y_prev is not None:
        in_specs.append(pl.BlockSpec(memory_space=pl.ANY))
        args.append(y_prev)
        kwargs["input_output_aliases"] = {5: 0}

    def body(ex_ref, x_ref, w1_ref, w3_ref, w2_ref, *rest):
        y_ref = rest[-1]
        _ffn_body(ex_ref, x_ref, w1_ref, w3_ref, w2_ref, y_ref, lo=lo)

    grid_spec = pltpu.PrefetchScalarGridSpec(
        num_scalar_prefetch=1,
        grid=(nt, 2),
        in_specs=in_specs,
        out_specs=pl.BlockSpec((TILE_M, D), lambda g, j, ex_s: (g + lo, 0)),
    )
    return pl.pallas_call(
        body,
        grid_spec=grid_spec,
        out_shape=jax.ShapeDtypeStruct((NSLOT, D), jnp.float32),
        **kwargs,
    )(*args)


def _combine_body(y_hbm, pos_hbm, rw_hbm, out_hbm, p0_v, p1_v, w_v,
                  bufs, gsems, wsems):
    T = out_hbm.shape[1]
    D = out_hbm.shape[2]
    nw = 32
    tpt = T // nw
    nch = 4
    ct = tpt // nch
    wid = lax.axis_index("s") * 2 + lax.axis_index("c")
    base = wid * tpt
    pltpu.sync_copy(pos_hbm.at[0, pl.ds(base, tpt)], p0_v)
    pltpu.sync_copy(pos_hbm.at[1, pl.ds(base, tpt)], p1_v)
    pltpu.sync_copy(rw_hbm.at[0, pl.ds(base, tpt)], w_v.at[0, pl.ds(0, tpt)])
    pltpu.sync_copy(rw_hbm.at[1, pl.ds(base, tpt)], w_v.at[1, pl.ds(0, tpt)])

    def g0(c):
        b = c % 2
        return pltpu.make_async_copy(
            y_hbm.at[p0_v.at[pl.ds(c * ct, ct)]], bufs[2 * b], gsems[2 * b])

    def g1(c):
        b = c % 2
        return pltpu.make_async_copy(
            y_hbm.at[p1_v.at[pl.ds(c * ct, ct)]], bufs[2 * b + 1],
            gsems[2 * b + 1])

    def wr(c):
        b = c % 2
        return pltpu.make_async_copy(
            bufs[2 * b], out_hbm.at[0, pl.ds(base + c * ct, ct)], wsems[b])

    g0(0).start()
    g1(0).start()
    g0(1).start()
    g1(1).start()
    for c in range(nch):
        g0(c).wait()
        g1(c).wait()
        b = c % 2
        buf0, buf1 = bufs[2 * b], bufs[2 * b + 1]

        def addrow(r, carry):
            tok = c * ct + r
            s0 = w_v[0, pl.ds(tok, 16)][0]
            s1 = w_v[1, pl.ds(tok, 16)][0]
            for cc in range(D // 16):
                col = cc * 16
                buf0[r, pl.ds(col, 16)] = (buf0[r, pl.ds(col, 16)] * s0
                                           + buf1[r, pl.ds(col, 16)] * s1)
            return carry
        lax.fori_loop(0, ct, addrow, 0)
        wr(c).start()
        if c + 2 < nch:
            wr(c).wait()
            g0(c + 2).start()
            g1(c + 2).start()
    wr(nch - 2).wait()
    wr(nch - 1).wait()


def _combine(y, pos, rw, T, D):
    mesh = plsc.VectorSubcoreMesh(core_axis_name="c", subcore_axis_name="s")
    ct = T // 32 // 4
    return pl.kernel(
        _combine_body,
        out_type=jax.ShapeDtypeStruct((1, T, D), jnp.float32),
        mesh=mesh,
        compiler_params=pltpu.CompilerParams(needs_layout_passes=False),
        scratch_types=[
            pltpu.VMEM((T // 32,), jnp.int32),
            pltpu.VMEM((T // 32,), jnp.int32),
            pltpu.VMEM((2, T // 32 + 16), jnp.float32),
            [pltpu.VMEM((ct, D), jnp.float32) for _ in range(4)],
            [pltpu.SemaphoreType.DMA for _ in range(4)],
            [pltpu.SemaphoreType.DMA for _ in range(2)],
        ],
    )(y, pos, rw)


def kernel(hidden_states, gate_w, w1, w3, w2):
    orig_shape = hidden_states.shape
    D = orig_shape[-1]
    x = hidden_states.reshape(-1, D)
    T = x.shape[0]
    pos, rw, ex, used = _router(x, gate_w)
    xs_a, gidx = _dispatch_a(x, pos, used)
    xs_b = _dispatch_b(x, gidx, used)
    exf = ex.reshape(32)
    y = _ffn_half(exf, xs_a, w1, w3, w2, 0)
    y = _ffn_half(exf, xs_b, w1, w3, w2, NUM_TILES // 2, y_prev=y)
    out = _combine(y, pos, rw, T, D)
    return out.reshape(orig_shape)


# speedup vs baseline: 1.0509x; 1.0509x over previous
import functools

import jax
import jax.numpy as jnp
from jax import lax
from jax.experimental import pallas as pl
from jax.experimental.pallas import tpu as pltpu
from jax.experimental.pallas import tpu_sc as plsc

TOPK = 2
TILE_M = 128
TILE_SHIFT = 7
NUM_TILES = 40
NSLOT = NUM_TILES * TILE_M
EX_W = 64
SCAN_CHUNK = 512


def _router_body(x_ref, gw_ref, pos_ref, rw_ref, ex_ref, used_ref):
    x = x_ref[...]
    gw = gw_ref[...]
    E = gw.shape[0]
    T = x.shape[0]
    logits = lax.dot_general(gw, x, (((1,), (1,)), ((), ())),
                             preferred_element_type=jnp.float32)
    row = lax.broadcasted_iota(jnp.int32, (E, T), 0)
    v0 = jnp.max(logits, axis=0, keepdims=True)
    a0 = jnp.min(jnp.where(logits == v0, row, E), axis=0, keepdims=True)
    masked = jnp.where(row == a0, -jnp.inf, logits)
    v1 = jnp.max(masked, axis=0, keepdims=True)
    a1 = jnp.min(jnp.where(masked == v1, row, E), axis=0, keepdims=True)
    d = jnp.exp(v1 - v0)
    w0 = 1.0 / (1.0 + d)
    w1 = d / (1.0 + d)

    oh0 = (row == a0).astype(jnp.float32)
    oh1 = (row == a1).astype(jnp.float32)

    C = SCAN_CHUNK
    ci = lax.broadcasted_iota(jnp.int32, (C, C), 0)
    cj = lax.broadcasted_iota(jnp.int32, (C, C), 1)
    upper_incl = (ci <= cj).astype(jnp.float32)
    carry = jnp.zeros((E, 1), jnp.float32)
    ranks = []
    for oh in (oh0, oh1):
        chunks = []
        for c in range(T // C):
            ohc = oh[:, c * C:(c + 1) * C]
            run = lax.dot_general(ohc, upper_incl, (((1,), (0,)), ((), ())),
                                  preferred_element_type=jnp.float32) + carry
            chunks.append(jnp.sum(run * ohc, axis=0, keepdims=True))
            carry = run[:, C - 1:C]
        ranks.append(jnp.concatenate(chunks, axis=1) - 1.0)
    counts = carry

    counts_i = counts.astype(jnp.int32)
    tiles = lax.shift_right_logical(counts_i + (TILE_M - 1), TILE_SHIFT)
    tiles_f = tiles.astype(jnp.float32)
    ei = lax.broadcasted_iota(jnp.int32, (E, E), 0)
    ej = lax.broadcasted_iota(jnp.int32, (E, E), 1)
    strict_lower = (ej < ei).astype(jnp.float32)
    tbase = lax.dot_general(strict_lower, tiles_f, (((1,), (0,)), ((), ())),
                            preferred_element_type=jnp.float32)
    pbase = tbase * float(TILE_M)

    pos0 = jnp.sum(oh0 * pbase, axis=0, keepdims=True) + ranks[0]
    pos1 = jnp.sum(oh1 * pbase, axis=0, keepdims=True) + ranks[1]
    pos_ref[0:1, :] = pos0.astype(jnp.int32)
    pos_ref[1:2, :] = pos1.astype(jnp.int32)
    rw_ref[0:1, :] = w0
    rw_ref[1:2, :] = w1

    g = lax.broadcasted_iota(jnp.int32, (1, EX_W), 1)
    owner = jnp.sum((tbase <= g.astype(jnp.float32)).astype(jnp.float32),
                    axis=0, keepdims=True).astype(jnp.int32) - 1
    total = jnp.sum(tiles_f).astype(jnp.int32)
    ex_ref[...] = jnp.where(g < total, owner, -1)
    used_ref[...] = jnp.zeros((1, 16), jnp.int32) + total * TILE_M


def _router(x, gate_w):
    T, D = x.shape
    return pl.pallas_call(
        _router_body,
        out_shape=(
            jax.ShapeDtypeStruct((2, T), jnp.int32),
            jax.ShapeDtypeStruct((2, T), jnp.float32),
            jax.ShapeDtypeStruct((1, EX_W), jnp.int32),
            jax.ShapeDtypeStruct((1, 16), jnp.int32),
        ),
    )(x, gate_w)


HALF = NSLOT // 2


def _pipelined_gather(x_hbm, dst_hbm, idx_ref, dst_base, idx_base, used,
                      glob_base, ch, rows0, rows1, sg0, sg1, sw0, sw1):
    def copy_in(c, buf, sem):
        idx = idx_ref.at[pl.ds(idx_base + c * ch, ch)]
        return pltpu.make_async_copy(x_hbm.at[idx], buf, sem)

    def copy_out(c, buf, sem):
        return pltpu.make_async_copy(
            buf, dst_hbm.at[pl.ds(dst_base + c * ch, ch)], sem)

    live = [glob_base + c * ch < used for c in range(2)]
    pl.when(live[0])(lambda: copy_in(0, rows0, sg0).start())
    pl.when(live[1])(lambda: copy_in(1, rows1, sg1).start())

    def drain0():
        copy_in(0, rows0, sg0).wait()
        copy_out(0, rows0, sw0).start()
    pl.when(live[0])(drain0)

    def drain1():
        copy_in(1, rows1, sg1).wait()
        copy_out(1, rows1, sw1).start()
    pl.when(live[1])(drain1)
    pl.when(live[0])(lambda: copy_out(0, rows0, sw0).wait())
    pl.when(live[1])(lambda: copy_out(1, rows1, sw1).wait())


def _dispatch_a_body(x_hbm, pos_hbm, used_hbm,
                     xs_hbm, gidx_hbm,
                     pos_v, used_v, gidx_v, rows0, rows1,
                     sg0, sg1, sw0, sw1):
    T, D = x_hbm.shape
    nw = 32
    spt = NSLOT // nw
    spa = HALF // nw
    ch = spa // 2
    wid = lax.axis_index("s") * 2 + lax.axis_index("c")
    base = wid * spt
    basea = wid * spa

    with jax.named_scope("disp_meta"):
        pltpu.sync_copy(pos_hbm, pos_v)
        pltpu.sync_copy(used_hbm, used_v)

        def init(i, c):
            off = base + i * 16
            gidx_v[pl.ds(off, 16)] = jnp.bitwise_and(
                lax.iota(jnp.int32, 16) + off, T - 1)
            offa = basea + i * 16
            gidx_v[pl.ds(offa, 16)] = jnp.bitwise_and(
                lax.iota(jnp.int32, 16) + offa, T - 1)
            return c
        lax.fori_loop(0, spt // 16, init, 0)

        def scat(j, c):
            tid = lax.iota(jnp.int32, 16) + j * 16
            for k in range(TOPK):
                p = pos_v[k, pl.ds(j * 16, 16)]
                plsc.store_scatter(gidx_v, [p], tid)
            return c
        lax.fori_loop(0, T // 16, scat, 0)

        pltpu.sync_copy(gidx_v.at[pl.ds(base, spt)], gidx_hbm.at[pl.ds(base, spt)])

    used = used_v[0, pl.ds(0, 16)][0]
    with jax.named_scope("disp_gather"):
        _pipelined_gather(x_hbm, xs_hbm, gidx_v, basea, basea, used,
                          basea, ch, rows0, rows1, sg0, sg1, sw0, sw1)


def _dispatch_a(x, pos, used):
    T, D = x.shape
    mesh = plsc.VectorSubcoreMesh(core_axis_name="c", subcore_axis_name="s")
    ch = HALF // 32 // 2
    return pl.kernel(
        _dispatch_a_body,
        out_type=(
            jax.ShapeDtypeStruct((HALF, D), jnp.float32),
            jax.ShapeDtypeStruct((NSLOT,), jnp.int32),
        ),
        mesh=mesh,
        compiler_params=pltpu.CompilerParams(needs_layout_passes=False),
        scratch_types=[
            pltpu.VMEM((2, T), jnp.int32),
            pltpu.VMEM((1, 16), jnp.int32),
            pltpu.VMEM((NSLOT,), jnp.int32),
            pltpu.VMEM((ch, D), jnp.float32),
            pltpu.VMEM((ch, D), jnp.float32),
            pltpu.SemaphoreType.DMA,
            pltpu.SemaphoreType.DMA,
            pltpu.SemaphoreType.DMA,
            pltpu.SemaphoreType.DMA,
        ],
    )(x, pos, used)


def _dispatch_b_body(x_hbm, gidx_hbm, used_hbm, xs_hbm,
                     gixb_v, used_v, rows0, rows1, sg0, sg1, sw0, sw1):
    T, D = x_hbm.shape
    nw = 32
    spb = HALF // nw
    ch = spb // 2
    wid = lax.axis_index("s") * 2 + lax.axis_index("c")
    base = wid * spb
    gbase = HALF + base

    pltpu.sync_copy(gidx_hbm.at[pl.ds(gbase, spb)], gixb_v)
    pltpu.sync_copy(used_hbm, used_v)
    used = used_v[0, pl.ds(0, 16)][0]
    _pipelined_gather(x_hbm, xs_hbm, gixb_v, base, 0, used,
                      gbase, ch, rows0, rows1, sg0, sg1, sw0, sw1)


def _dispatch_b(x, gidx, used):
    T, D = x.shape
    mesh = plsc.VectorSubcoreMesh(core_axis_name="c", subcore_axis_name="s")
    ch = HALF // 32 // 2
    return pl.kernel(
        _dispatch_b_body,
        out_type=jax.ShapeDtypeStruct((HALF, D), jnp.float32),
        mesh=mesh,
        compiler_params=pltpu.CompilerParams(needs_layout_passes=False),
        scratch_types=[
            pltpu.VMEM((HALF // 32,), jnp.int32),
            pltpu.VMEM((1, 16), jnp.int32),
            pltpu.VMEM((ch, D), jnp.float32),
            pltpu.VMEM((ch, D), jnp.float32),
            pltpu.SemaphoreType.DMA,
            pltpu.SemaphoreType.DMA,
            pltpu.SemaphoreType.DMA,
            pltpu.SemaphoreType.DMA,
        ],
    )(x, gidx, used)


def _ffn_body(ex_ref, x_ref, w1_ref, w3_ref, w2_ref, y_ref, lo=0):
    @pl.when(ex_ref[pl.program_id(0) + lo] >= 0)
    def _():
        x = x_ref[...]
        g = lax.dot_general(x, w1_ref[0], (((1,), (1,)), ((), ())),
                            preferred_element_type=jnp.float32)
        u = lax.dot_general(x, w3_ref[0], (((1,), (1,)), ((), ())),
                            preferred_element_type=jnp.float32)
        h = g * jax.nn.sigmoid(g) * u
        y_ref[...] = lax.dot_general(h, w2_ref[0], (((1,), (1,)), ((), ())),
                                     preferred_element_type=jnp.float32)


def _ffn_half(ex, xs_half, w1, w3, w2, lo, y_prev=None):
    E, F, D = w1.shape
    nt = NUM_TILES // 2

    def wsel(g, ex_s):
        return (jnp.maximum(ex_s[g + lo], 0), 0, 0)

    in_specs = [
        pl.BlockSpec((TILE_M, D), lambda g, ex_s: (g, 0)),
        pl.BlockSpec((1, F, D), wsel),
        pl.BlockSpec((1, F, D), wsel),
        pl.BlockSpec((1, D, F), wsel),
    ]
    args = [ex, xs_half, w1, w3, w2]
    kwargs = {}
    if y_prev is not None:
        in_specs.append(pl.BlockSpec(memory_space=pl.ANY))
        args.append(y_prev)
        kwargs["input_output_aliases"] = {5: 0}

    def body(ex_ref, x_ref, w1_ref, w3_ref, w2_ref, *rest):
        y_ref = rest[-1]
        _ffn_body(ex_ref, x_ref, w1_ref, w3_ref, w2_ref, y_ref, lo=lo)

    grid_spec = pltpu.PrefetchScalarGridSpec(
        num_scalar_prefetch=1,
        grid=(nt,),
        in_specs=in_specs,
        out_specs=pl.BlockSpec((TILE_M, D), lambda g, ex_s: (g + lo, 0)),
    )
    return pl.pallas_call(
        body,
        grid_spec=grid_spec,
        out_shape=jax.ShapeDtypeStruct((NSLOT, D), jnp.float32),
        **kwargs,
    )(*args)


def _combine_body(y_hbm, pos_hbm, rw_hbm, out_hbm, p0_v, p1_v, w_v,
                  bufs, gsems, wsems):
    T = out_hbm.shape[1]
    D = out_hbm.shape[2]
    nw = 32
    tpt = T // nw
    nch = 4
    ct = tpt // nch
    wid = lax.axis_index("s") * 2 + lax.axis_index("c")
    base = wid * tpt
    pltpu.sync_copy(pos_hbm.at[0, pl.ds(base, tpt)], p0_v)
    pltpu.sync_copy(pos_hbm.at[1, pl.ds(base, tpt)], p1_v)
    pltpu.sync_copy(rw_hbm.at[0, pl.ds(base, tpt)], w_v.at[0, pl.ds(0, tpt)])
    pltpu.sync_copy(rw_hbm.at[1, pl.ds(base, tpt)], w_v.at[1, pl.ds(0, tpt)])

    def g0(c):
        b = c % 2
        return pltpu.make_async_copy(
            y_hbm.at[p0_v.at[pl.ds(c * ct, ct)]], bufs[2 * b], gsems[2 * b])

    def g1(c):
        b = c % 2
        return pltpu.make_async_copy(
            y_hbm.at[p1_v.at[pl.ds(c * ct, ct)]], bufs[2 * b + 1],
            gsems[2 * b + 1])

    def wr(c):
        b = c % 2
        return pltpu.make_async_copy(
            bufs[2 * b], out_hbm.at[0, pl.ds(base + c * ct, ct)], wsems[b])

    g0(0).start()
    g1(0).start()
    g0(1).start()
    g1(1).start()
    for c in range(nch):
        g0(c).wait()
        g1(c).wait()
        b = c % 2
        buf0, buf1 = bufs[2 * b], bufs[2 * b + 1]

        def addrow(r, carry):
            tok = c * ct + r
            s0 = w_v[0, pl.ds(tok, 16)][0]
            s1 = w_v[1, pl.ds(tok, 16)][0]
            for cc in range(D // 16):
                col = cc * 16
                buf0[r, pl.ds(col, 16)] = (buf0[r, pl.ds(col, 16)] * s0
                                           + buf1[r, pl.ds(col, 16)] * s1)
            return carry
        lax.fori_loop(0, ct, addrow, 0)
        wr(c).start()
        if c + 2 < nch:
            wr(c).wait()
            g0(c + 2).start()
            g1(c + 2).start()
    wr(nch - 2).wait()
    wr(nch - 1).wait()


def _combine(y, pos, rw, T, D):
    mesh = plsc.VectorSubcoreMesh(core_axis_name="c", subcore_axis_name="s")
    ct = T // 32 // 4
    return pl.kernel(
        _combine_body,
        out_type=jax.ShapeDtypeStruct((1, T, D), jnp.float32),
        mesh=mesh,
        compiler_params=pltpu.CompilerParams(needs_layout_passes=False),
        scratch_types=[
            pltpu.VMEM((T // 32,), jnp.int32),
            pltpu.VMEM((T // 32,), jnp.int32),
            pltpu.VMEM((2, T // 32 + 16), jnp.float32),
            [pltpu.VMEM((ct, D), jnp.float32) for _ in range(4)],
            [pltpu.SemaphoreType.DMA for _ in range(4)],
            [pltpu.SemaphoreType.DMA for _ in range(2)],
        ],
    )(y, pos, rw)


def kernel(hidden_states, gate_w, w1, w3, w2):
    orig_shape = hidden_states.shape
    D = orig_shape[-1]
    x = hidden_states.reshape(-1, D)
    T = x.shape[0]
    pos, rw, ex, used = _router(x, gate_w)
    xs_a, gidx = _dispatch_a(x, pos, used)
    xs_b = _dispatch_b(x, gidx, used)
    exf = ex.reshape(EX_W)
    y = _ffn_half(exf, xs_a, w1, w3, w2, 0)
    y = _ffn_half(exf, xs_b, w1, w3, w2, NUM_TILES // 2, y_prev=y)
    out = _combine(y, pos, rw, T, D)
    return out.reshape(orig_shape)


# speedup vs baseline: 1.2442x; 1.1840x over previous
import functools

import jax
import jax.numpy as jnp
from jax import lax
from jax.experimental import pallas as pl
from jax.experimental.pallas import tpu as pltpu
from jax.experimental.pallas import tpu_sc as plsc

TOPK = 2
TILE_M = 256
TILE_SHIFT = 8
NUM_TILES = 24
NSLOT = NUM_TILES * TILE_M
EX_W = 32
SCAN_CHUNK = 512


def _router_body(x_ref, gw_ref, pos_ref, rw_ref, ex_ref, used_ref):
    x = x_ref[...]
    gw = gw_ref[...]
    E = gw.shape[0]
    T = x.shape[0]
    logits = lax.dot_general(gw, x, (((1,), (1,)), ((), ())),
                             preferred_element_type=jnp.float32)
    row = lax.broadcasted_iota(jnp.int32, (E, T), 0)
    v0 = jnp.max(logits, axis=0, keepdims=True)
    a0 = jnp.min(jnp.where(logits == v0, row, E), axis=0, keepdims=True)
    masked = jnp.where(row == a0, -jnp.inf, logits)
    v1 = jnp.max(masked, axis=0, keepdims=True)
    a1 = jnp.min(jnp.where(masked == v1, row, E), axis=0, keepdims=True)
    d = jnp.exp(v1 - v0)
    w0 = 1.0 / (1.0 + d)
    w1 = d / (1.0 + d)

    oh0 = (row == a0).astype(jnp.float32)
    oh1 = (row == a1).astype(jnp.float32)

    C = SCAN_CHUNK
    ci = lax.broadcasted_iota(jnp.int32, (C, C), 0)
    cj = lax.broadcasted_iota(jnp.int32, (C, C), 1)
    upper_incl = (ci <= cj).astype(jnp.float32)
    carry = jnp.zeros((E, 1), jnp.float32)
    ranks = []
    for oh in (oh0, oh1):
        chunks = []
        for c in range(T // C):
            ohc = oh[:, c * C:(c + 1) * C]
            run = lax.dot_general(ohc, upper_incl, (((1,), (0,)), ((), ())),
                                  preferred_element_type=jnp.float32) + carry
            chunks.append(jnp.sum(run * ohc, axis=0, keepdims=True))
            carry = run[:, C - 1:C]
        ranks.append(jnp.concatenate(chunks, axis=1) - 1.0)
    counts = carry

    counts_i = counts.astype(jnp.int32)
    tiles = lax.shift_right_logical(counts_i + (TILE_M - 1), TILE_SHIFT)
    tiles_f = tiles.astype(jnp.float32)
    ei = lax.broadcasted_iota(jnp.int32, (E, E), 0)
    ej = lax.broadcasted_iota(jnp.int32, (E, E), 1)
    strict_lower = (ej < ei).astype(jnp.float32)
    tbase = lax.dot_general(strict_lower, tiles_f, (((1,), (0,)), ((), ())),
                            preferred_element_type=jnp.float32)
    pbase = tbase * float(TILE_M)

    pos0 = jnp.sum(oh0 * pbase, axis=0, keepdims=True) + ranks[0]
    pos1 = jnp.sum(oh1 * pbase, axis=0, keepdims=True) + ranks[1]
    pos_ref[0:1, :] = pos0.astype(jnp.int32)
    pos_ref[1:2, :] = pos1.astype(jnp.int32)
    rw_ref[0:1, :] = w0
    rw_ref[1:2, :] = w1

    g = lax.broadcasted_iota(jnp.int32, (1, EX_W), 1)
    owner = jnp.sum((tbase <= g.astype(jnp.float32)).astype(jnp.float32),
                    axis=0, keepdims=True).astype(jnp.int32) - 1
    total = jnp.sum(tiles_f).astype(jnp.int32)
    ex_ref[...] = jnp.where(g < total, owner, -1)
    used_ref[...] = jnp.zeros((1, 16), jnp.int32) + total * TILE_M


def _router(x, gate_w):
    T, D = x.shape
    return pl.pallas_call(
        _router_body,
        out_shape=(
            jax.ShapeDtypeStruct((2, T), jnp.int32),
            jax.ShapeDtypeStruct((2, T), jnp.float32),
            jax.ShapeDtypeStruct((1, EX_W), jnp.int32),
            jax.ShapeDtypeStruct((1, 16), jnp.int32),
        ),
    )(x, gate_w)


HALF = NSLOT // 2


def _pipelined_gather(x_hbm, dst_hbm, idx_ref, dst_base, idx_base, used,
                      glob_base, ch, rows0, rows1, sg0, sg1, sw0, sw1):
    def copy_in(c, buf, sem):
        idx = idx_ref.at[pl.ds(idx_base + c * ch, ch)]
        return pltpu.make_async_copy(x_hbm.at[idx], buf, sem)

    def copy_out(c, buf, sem):
        return pltpu.make_async_copy(
            buf, dst_hbm.at[pl.ds(dst_base + c * ch, ch)], sem)

    live = [glob_base + c * ch < used for c in range(2)]
    pl.when(live[0])(lambda: copy_in(0, rows0, sg0).start())
    pl.when(live[1])(lambda: copy_in(1, rows1, sg1).start())

    def drain0():
        copy_in(0, rows0, sg0).wait()
        copy_out(0, rows0, sw0).start()
    pl.when(live[0])(drain0)

    def drain1():
        copy_in(1, rows1, sg1).wait()
        copy_out(1, rows1, sw1).start()
    pl.when(live[1])(drain1)
    pl.when(live[0])(lambda: copy_out(0, rows0, sw0).wait())
    pl.when(live[1])(lambda: copy_out(1, rows1, sw1).wait())


def _dispatch_a_body(x_hbm, pos_hbm, used_hbm,
                     xs_hbm, gidx_hbm,
                     pos_v, used_v, gidx_v, rows0, rows1,
                     sg0, sg1, sw0, sw1):
    T, D = x_hbm.shape
    nw = 32
    spt = NSLOT // nw
    spa = HALF // nw
    ch = spa // 2
    wid = lax.axis_index("s") * 2 + lax.axis_index("c")
    base = wid * spt
    basea = wid * spa

    with jax.named_scope("disp_meta"):
        pltpu.sync_copy(pos_hbm, pos_v)
        pltpu.sync_copy(used_hbm, used_v)

        def init(i, c):
            off = base + i * 16
            gidx_v[pl.ds(off, 16)] = jnp.bitwise_and(
                lax.iota(jnp.int32, 16) + off, T - 1)
            offa = basea + i * 16
            gidx_v[pl.ds(offa, 16)] = jnp.bitwise_and(
                lax.iota(jnp.int32, 16) + offa, T - 1)
            return c
        lax.fori_loop(0, spt // 16, init, 0)

        def scat(j, c):
            tid = lax.iota(jnp.int32, 16) + j * 16
            for k in range(TOPK):
                p = pos_v[k, pl.ds(j * 16, 16)]
                plsc.store_scatter(gidx_v, [p], tid)
            return c
        lax.fori_loop(0, T // 16, scat, 0)

        pltpu.sync_copy(gidx_v.at[pl.ds(base, spt)], gidx_hbm.at[pl.ds(base, spt)])

    used = used_v[0, pl.ds(0, 16)][0]
    with jax.named_scope("disp_gather"):
        _pipelined_gather(x_hbm, xs_hbm, gidx_v, basea, basea, used,
                          basea, ch, rows0, rows1, sg0, sg1, sw0, sw1)


def _dispatch_a(x, pos, used):
    T, D = x.shape
    mesh = plsc.VectorSubcoreMesh(core_axis_name="c", subcore_axis_name="s")
    ch = HALF // 32 // 2
    return pl.kernel(
        _dispatch_a_body,
        out_type=(
            jax.ShapeDtypeStruct((HALF, D), jnp.float32),
            jax.ShapeDtypeStruct((NSLOT,), jnp.int32),
        ),
        mesh=mesh,
        compiler_params=pltpu.CompilerParams(needs_layout_passes=False),
        scratch_types=[
            pltpu.VMEM((2, T), jnp.int32),
            pltpu.VMEM((1, 16), jnp.int32),
            pltpu.VMEM((NSLOT,), jnp.int32),
            pltpu.VMEM((ch, D), jnp.float32),
            pltpu.VMEM((ch, D), jnp.float32),
            pltpu.SemaphoreType.DMA,
            pltpu.SemaphoreType.DMA,
            pltpu.SemaphoreType.DMA,
            pltpu.SemaphoreType.DMA,
        ],
    )(x, pos, used)


def _dispatch_b_body(x_hbm, gidx_hbm, used_hbm, xs_hbm,
                     gixb_v, used_v, rows0, rows1, sg0, sg1, sw0, sw1):
    T, D = x_hbm.shape
    nw = 32
    spb = HALF // nw
    ch = spb // 2
    wid = lax.axis_index("s") * 2 + lax.axis_index("c")
    base = wid * spb
    gbase = HALF + base

    pltpu.sync_copy(gidx_hbm.at[pl.ds(gbase, spb)], gixb_v)
    pltpu.sync_copy(used_hbm, used_v)
    used = used_v[0, pl.ds(0, 16)][0]
    _pipelined_gather(x_hbm, xs_hbm, gixb_v, base, 0, used,
                      gbase, ch, rows0, rows1, sg0, sg1, sw0, sw1)


def _dispatch_b(x, gidx, used):
    T, D = x.shape
    mesh = plsc.VectorSubcoreMesh(core_axis_name="c", subcore_axis_name="s")
    ch = HALF // 32 // 2
    return pl.kernel(
        _dispatch_b_body,
        out_type=jax.ShapeDtypeStruct((HALF, D), jnp.float32),
        mesh=mesh,
        compiler_params=pltpu.CompilerParams(needs_layout_passes=False),
        scratch_types=[
            pltpu.VMEM((HALF // 32,), jnp.int32),
            pltpu.VMEM((1, 16), jnp.int32),
            pltpu.VMEM((ch, D), jnp.float32),
            pltpu.VMEM((ch, D), jnp.float32),
            pltpu.SemaphoreType.DMA,
            pltpu.SemaphoreType.DMA,
            pltpu.SemaphoreType.DMA,
            pltpu.SemaphoreType.DMA,
        ],
    )(x, gidx, used)


def _ffn_body(ex_ref, x_ref, w1_ref, w3_ref, w2_ref, y_ref, lo=0):
    @pl.when(ex_ref[pl.program_id(0) + lo] >= 0)
    def _():
        x = x_ref[...]
        g = lax.dot_general(x, w1_ref[0], (((1,), (1,)), ((), ())),
                            preferred_element_type=jnp.float32)
        u = lax.dot_general(x, w3_ref[0], (((1,), (1,)), ((), ())),
                            preferred_element_type=jnp.float32)
        h = g * jax.nn.sigmoid(g) * u
        y_ref[...] = lax.dot_general(h, w2_ref[0], (((1,), (1,)), ((), ())),
                                     preferred_element_type=jnp.float32)


def _ffn_half(ex, xs_half, w1, w3, w2, lo, y_prev=None):
    E, F, D = w1.shape
    nt = NUM_TILES // 2

    def wsel(g, ex_s):
        return (jnp.maximum(ex_s[g + lo], 0), 0, 0)

    in_specs = [
        pl.BlockSpec((TILE_M, D), lambda g, ex_s: (g, 0)),
        pl.BlockSpec((1, F, D), wsel),
        pl.BlockSpec((1, F, D), wsel),
        pl.BlockSpec((1, D, F), wsel),
    ]
    args = [ex, xs_half, w1, w3, w2]
    kwargs = {}
    if y_prev is not None:
        in_specs.append(pl.BlockSpec(memory_space=pl.ANY))
        args.append(y_prev)
        kwargs["input_output_aliases"] = {5: 0}

    def body(ex_ref, x_ref, w1_ref, w3_ref, w2_ref, *rest):
        y_ref = rest[-1]
        _ffn_body(ex_ref, x_ref, w1_ref, w3_ref, w2_ref, y_ref, lo=lo)

    grid_spec = pltpu.PrefetchScalarGridSpec(
        num_scalar_prefetch=1,
        grid=(nt,),
        in_specs=in_specs,
        out_specs=pl.BlockSpec((TILE_M, D), lambda g, ex_s: (g + lo, 0)),
    )
    return pl.pallas_call(
        body,
        grid_spec=grid_spec,
        out_shape=jax.ShapeDtypeStruct((NSLOT, D), jnp.float32),
        **kwargs,
    )(*args)


def _combine_body(y_hbm, pos_hbm, rw_hbm, out_hbm, p0_v, p1_v, w_v,
                  bufs, gsems, wsems):
    T = out_hbm.shape[1]
    D = out_hbm.shape[2]
    nw = 32
    tpt = T // nw
    nch = 4
    ct = tpt // nch
    wid = lax.axis_index("s") * 2 + lax.axis_index("c")
    base = wid * tpt
    pltpu.sync_copy(pos_hbm.at[0, pl.ds(base, tpt)], p0_v)
    pltpu.sync_copy(pos_hbm.at[1, pl.ds(base, tpt)], p1_v)
    pltpu.sync_copy(rw_hbm.at[0, pl.ds(base, tpt)], w_v.at[0, pl.ds(0, tpt)])
    pltpu.sync_copy(rw_hbm.at[1, pl.ds(base, tpt)], w_v.at[1, pl.ds(0, tpt)])

    def g0(c):
        b = c % 2
        return pltpu.make_async_copy(
            y_hbm.at[p0_v.at[pl.ds(c * ct, ct)]], bufs[2 * b], gsems[2 * b])

    def g1(c):
        b = c % 2
        return pltpu.make_async_copy(
            y_hbm.at[p1_v.at[pl.ds(c * ct, ct)]], bufs[2 * b + 1],
            gsems[2 * b + 1])

    def wr(c):
        b = c % 2
        return pltpu.make_async_copy(
            bufs[2 * b], out_hbm.at[0, pl.ds(base + c * ct, ct)], wsems[b])

    g0(0).start()
    g1(0).start()
    g0(1).start()
    g1(1).start()
    for c in range(nch):
        g0(c).wait()
        g1(c).wait()
        b = c % 2
        buf0, buf1 = bufs[2 * b], bufs[2 * b + 1]

        def addrow(r, carry):
            tok = c * ct + r
            s0 = w_v[0, pl.ds(tok, 16)][0]
            s1 = w_v[1, pl.ds(tok, 16)][0]
            for cc in range(D // 16):
                col = cc * 16
                buf0[r, pl.ds(col, 16)] = (buf0[r, pl.ds(col, 16)] * s0
                                           + buf1[r, pl.ds(col, 16)] * s1)
            return carry
        lax.fori_loop(0, ct, addrow, 0)
        wr(c).start()
        if c + 2 < nch:
            wr(c).wait()
            g0(c + 2).start()
            g1(c + 2).start()
    wr(nch - 2).wait()
    wr(nch - 1).wait()


def _combine(y, pos, rw, T, D):
    mesh = plsc.VectorSubcoreMesh(core_axis_name="c", subcore_axis_name="s")
    ct = T // 32 // 4
    return pl.kernel(
        _combine_body,
        out_type=jax.ShapeDtypeStruct((1, T, D), jnp.float32),
        mesh=mesh,
        compiler_params=pltpu.CompilerParams(needs_layout_passes=False),
        scratch_types=[
            pltpu.VMEM((T // 32,), jnp.int32),
            pltpu.VMEM((T // 32,), jnp.int32),
            pltpu.VMEM((2, T // 32 + 16), jnp.float32),
            [pltpu.VMEM((ct, D), jnp.float32) for _ in range(4)],
            [pltpu.SemaphoreType.DMA for _ in range(4)],
            [pltpu.SemaphoreType.DMA for _ in range(2)],
        ],
    )(y, pos, rw)


def kernel(hidden_states, gate_w, w1, w3, w2):
    orig_shape = hidden_states.shape
    D = orig_shape[-1]
    x = hidden_states.reshape(-1, D)
    T = x.shape[0]
    pos, rw, ex, used = _router(x, gate_w)
    xs_a, gidx = _dispatch_a(x, pos, used)
    xs_b = _dispatch_b(x, gidx, used)
    exf = ex.reshape(EX_W)
    y = _ffn_half(exf, xs_a, w1, w3, w2, 0)
    y = _ffn_half(exf, xs_b, w1, w3, w2, NUM_TILES // 2, y_prev=y)
    out = _combine(y, pos, rw, T, D)
    return out.reshape(orig_shape)


# speedup vs baseline: 1.2741x; 1.0241x over previous
import functools

import jax
import jax.numpy as jnp
from jax import lax
from jax.experimental import pallas as pl
from jax.experimental.pallas import tpu as pltpu
from jax.experimental.pallas import tpu_sc as plsc

TOPK = 2
TILE_M = 256
TILE_SHIFT = 8
NUM_TILES = 24
NSLOT = NUM_TILES * TILE_M
EX_W = 32
SCAN_CHUNK = 512


def _router_body(x_ref, gw_ref, pos_ref, rw_ref, ex_ref, used_ref):
    x = x_ref[...]
    gw = gw_ref[...]
    E = gw.shape[0]
    T = x.shape[0]
    logits = lax.dot_general(gw, x, (((1,), (1,)), ((), ())),
                             preferred_element_type=jnp.float32)
    row = lax.broadcasted_iota(jnp.int32, (E, T), 0)
    v0 = jnp.max(logits, axis=0, keepdims=True)
    a0 = jnp.min(jnp.where(logits == v0, row, E), axis=0, keepdims=True)
    masked = jnp.where(row == a0, -jnp.inf, logits)
    v1 = jnp.max(masked, axis=0, keepdims=True)
    a1 = jnp.min(jnp.where(masked == v1, row, E), axis=0, keepdims=True)
    d = jnp.exp(v1 - v0)
    w0 = 1.0 / (1.0 + d)
    w1 = d / (1.0 + d)

    oh0 = (row == a0).astype(jnp.float32)
    oh1 = (row == a1).astype(jnp.float32)

    C = SCAN_CHUNK
    ci = lax.broadcasted_iota(jnp.int32, (C, C), 0)
    cj = lax.broadcasted_iota(jnp.int32, (C, C), 1)
    upper_incl = (ci <= cj).astype(jnp.float32)
    carry = jnp.zeros((E, 1), jnp.float32)
    ranks = []
    for oh in (oh0, oh1):
        chunks = []
        for c in range(T // C):
            ohc = oh[:, c * C:(c + 1) * C]
            run = lax.dot_general(ohc, upper_incl, (((1,), (0,)), ((), ())),
                                  preferred_element_type=jnp.float32) + carry
            chunks.append(jnp.sum(run * ohc, axis=0, keepdims=True))
            carry = run[:, C - 1:C]
        ranks.append(jnp.concatenate(chunks, axis=1) - 1.0)
    counts = carry

    counts_i = counts.astype(jnp.int32)
    tiles = lax.shift_right_logical(counts_i + (TILE_M - 1), TILE_SHIFT)
    tiles_f = tiles.astype(jnp.float32)
    ei = lax.broadcasted_iota(jnp.int32, (E, E), 0)
    ej = lax.broadcasted_iota(jnp.int32, (E, E), 1)
    strict_lower = (ej < ei).astype(jnp.float32)
    tbase = lax.dot_general(strict_lower, tiles_f, (((1,), (0,)), ((), ())),
                            preferred_element_type=jnp.float32)
    pbase = tbase * float(TILE_M)

    pos0 = jnp.sum(oh0 * pbase, axis=0, keepdims=True) + ranks[0]
    pos1 = jnp.sum(oh1 * pbase, axis=0, keepdims=True) + ranks[1]
    pos_ref[0:1, :] = pos0.astype(jnp.int32)
    pos_ref[1:2, :] = pos1.astype(jnp.int32)
    rw_ref[0:1, :] = w0
    rw_ref[1:2, :] = w1

    g = lax.broadcasted_iota(jnp.int32, (1, EX_W), 1)
    owner = jnp.sum((tbase <= g.astype(jnp.float32)).astype(jnp.float32),
                    axis=0, keepdims=True).astype(jnp.int32) - 1
    total = jnp.sum(tiles_f).astype(jnp.int32)
    ex_ref[...] = jnp.where(g < total, owner, -1)
    used_ref[...] = jnp.zeros((1, 16), jnp.int32) + total * TILE_M


def _router(x, gate_w):
    T, D = x.shape
    return pl.pallas_call(
        _router_body,
        out_shape=(
            jax.ShapeDtypeStruct((2, T), jnp.int32),
            jax.ShapeDtypeStruct((2, T), jnp.float32),
            jax.ShapeDtypeStruct((1, EX_W), jnp.int32),
            jax.ShapeDtypeStruct((1, 16), jnp.int32),
        ),
    )(x, gate_w)


SLOTS_A = 4 * TILE_M
SLOTS_B = NSLOT - SLOTS_A


def _pipelined_gather(x_hbm, dst_hbm, idx_ref, dst_base, idx_base, used,
                      glob_base, ch, nch, rows, gsems, wsems):
    def copy_in(c):
        b = c % 2
        idx = idx_ref.at[pl.ds(idx_base + c * ch, ch)]
        return pltpu.make_async_copy(x_hbm.at[idx], rows[b], gsems[b])

    def copy_out(c):
        b = c % 2
        return pltpu.make_async_copy(
            rows[b], dst_hbm.at[pl.ds(dst_base + c * ch, ch)], wsems[b])

    live = [glob_base + c * ch < used for c in range(nch)]
    pl.when(live[0])(lambda: copy_in(0).start())
    if nch > 1:
        pl.when(live[1])(lambda: copy_in(1).start())
    for c in range(nch):
        def drain(c=c):
            copy_in(c).wait()
            copy_out(c).start()
        pl.when(live[c])(drain)
        if c + 2 < nch:
            pl.when(live[c])(lambda c=c: copy_out(c).wait())
            pl.when(live[c + 2])(lambda c=c: copy_in(c + 2).start())
    for c in range(max(0, nch - 2), nch):
        pl.when(live[c])(lambda c=c: copy_out(c).wait())


def _dispatch_a_body(x_hbm, pos_hbm, used_hbm,
                     xs_hbm, gidx_hbm,
                     pos_v, used_v, gidx_v, rows0, rows1,
                     sg0, sg1, sw0, sw1):
    T, D = x_hbm.shape
    nw = 32
    spt = NSLOT // nw
    spa = SLOTS_A // nw
    ch = spa // 2
    wid = lax.axis_index("s") * 2 + lax.axis_index("c")
    base = wid * spt
    basea = wid * spa

    with jax.named_scope("disp_meta"):
        pltpu.sync_copy(pos_hbm, pos_v)
        pltpu.sync_copy(used_hbm, used_v)

        def init(i, c):
            off = base + i * 16
            gidx_v[pl.ds(off, 16)] = jnp.bitwise_and(
                lax.iota(jnp.int32, 16) + off, T - 1)
            offa = basea + i * 16
            gidx_v[pl.ds(offa, 16)] = jnp.bitwise_and(
                lax.iota(jnp.int32, 16) + offa, T - 1)
            return c
        lax.fori_loop(0, spt // 16, init, 0)

        def scat(j, c):
            tid = lax.iota(jnp.int32, 16) + j * 16
            for k in range(TOPK):
                p = pos_v[k, pl.ds(j * 16, 16)]
                plsc.store_scatter(gidx_v, [p], tid)
            return c
        lax.fori_loop(0, T // 16, scat, 0)

        pltpu.sync_copy(gidx_v.at[pl.ds(base, spt)], gidx_hbm.at[pl.ds(base, spt)])

    used = used_v[0, pl.ds(0, 16)][0]
    with jax.named_scope("disp_gather"):
        _pipelined_gather(x_hbm, xs_hbm, gidx_v, basea, basea, used,
                          basea, ch, 2, (rows0, rows1), (sg0, sg1), (sw0, sw1))


def _dispatch_a(x, pos, used):
    T, D = x.shape
    mesh = plsc.VectorSubcoreMesh(core_axis_name="c", subcore_axis_name="s")
    ch = SLOTS_A // 32 // 2
    return pl.kernel(
        _dispatch_a_body,
        out_type=(
            jax.ShapeDtypeStruct((SLOTS_A, D), jnp.float32),
            jax.ShapeDtypeStruct((NSLOT,), jnp.int32),
        ),
        mesh=mesh,
        compiler_params=pltpu.CompilerParams(needs_layout_passes=False),
        scratch_types=[
            pltpu.VMEM((2, T), jnp.int32),
            pltpu.VMEM((1, 16), jnp.int32),
            pltpu.VMEM((NSLOT,), jnp.int32),
            pltpu.VMEM((ch, D), jnp.float32),
            pltpu.VMEM((ch, D), jnp.float32),
            pltpu.SemaphoreType.DMA,
            pltpu.SemaphoreType.DMA,
            pltpu.SemaphoreType.DMA,
            pltpu.SemaphoreType.DMA,
        ],
    )(x, pos, used)


def _dispatch_b_body(x_hbm, gidx_hbm, used_hbm, xs_hbm,
                     gixb_v, used_v, rows0, rows1, sg0, sg1, sw0, sw1):
    T, D = x_hbm.shape
    nw = 32
    spb = SLOTS_B // nw
    ch = spb // 4
    wid = lax.axis_index("s") * 2 + lax.axis_index("c")
    base = wid * spb
    gbase = SLOTS_A + base

    pltpu.sync_copy(gidx_hbm.at[pl.ds(gbase, spb)], gixb_v)
    pltpu.sync_copy(used_hbm, used_v)
    used = used_v[0, pl.ds(0, 16)][0]
    _pipelined_gather(x_hbm, xs_hbm, gixb_v, base, 0, used,
                      gbase, ch, 4, (rows0, rows1), (sg0, sg1), (sw0, sw1))


def _dispatch_b(x, gidx, used):
    T, D = x.shape
    mesh = plsc.VectorSubcoreMesh(core_axis_name="c", subcore_axis_name="s")
    ch = SLOTS_B // 32 // 4
    return pl.kernel(
        _dispatch_b_body,
        out_type=jax.ShapeDtypeStruct((SLOTS_B, D), jnp.float32),
        mesh=mesh,
        compiler_params=pltpu.CompilerParams(needs_layout_passes=False),
        scratch_types=[
            pltpu.VMEM((SLOTS_B // 32,), jnp.int32),
            pltpu.VMEM((1, 16), jnp.int32),
            pltpu.VMEM((ch, D), jnp.float32),
            pltpu.VMEM((ch, D), jnp.float32),
            pltpu.SemaphoreType.DMA,
            pltpu.SemaphoreType.DMA,
            pltpu.SemaphoreType.DMA,
            pltpu.SemaphoreType.DMA,
        ],
    )(x, gidx, used)


def _ffn_body(ex_ref, x_ref, w1_ref, w3_ref, w2_ref, y_ref, lo=0):
    @pl.when(ex_ref[pl.program_id(0) + lo] >= 0)
    def _():
        x = x_ref[...]
        g = lax.dot_general(x, w1_ref[0], (((1,), (1,)), ((), ())),
                            preferred_element_type=jnp.float32)
        u = lax.dot_general(x, w3_ref[0], (((1,), (1,)), ((), ())),
                            preferred_element_type=jnp.float32)
        h = g * jax.nn.sigmoid(g) * u
        y_ref[...] = lax.dot_general(h, w2_ref[0], (((1,), (1,)), ((), ())),
                                     preferred_element_type=jnp.float32)


def _ffn_half(ex, xs_half, w1, w3, w2, lo, y_prev=None):
    E, F, D = w1.shape
    nt = xs_half.shape[0] // TILE_M

    def wsel(g, ex_s):
        return (jnp.maximum(ex_s[g + lo], 0), 0, 0)

    in_specs = [
        pl.BlockSpec((TILE_M, D), lambda g, ex_s: (g, 0)),
        pl.BlockSpec((1, F, D), wsel),
        pl.BlockSpec((1, F, D), wsel),
        pl.BlockSpec((1, D, F), wsel),
    ]
    args = [ex, xs_half, w1, w3, w2]
    kwargs = {}
    if y_prev is not None:
        in_specs.append(pl.BlockSpec(memory_space=pl.ANY))
        args.append(y_prev)
        kwargs["input_output_aliases"] = {5: 0}

    def body(ex_ref, x_ref, w1_ref, w3_ref, w2_ref, *rest):
        y_ref = rest[-1]
        _ffn_body(ex_ref, x_ref, w1_ref, w3_ref, w2_ref, y_ref, lo=lo)

    grid_spec = pltpu.PrefetchScalarGridSpec(
        num_scalar_prefetch=1,
        grid=(nt,),
        in_specs=in_specs,
        out_specs=pl.BlockSpec((TILE_M, D), lambda g, ex_s: (g + lo, 0)),
    )
    return pl.pallas_call(
        body,
        grid_spec=grid_spec,
        out_shape=jax.ShapeDtypeStruct((NSLOT, D), jnp.float32),
        **kwargs,
    )(*args)


def _combine_body(y_hbm, pos_hbm, rw_hbm, out_hbm, p0_v, p1_v, w_v,
                  bufs, gsems, wsems):
    T = out_hbm.shape[1]
    D = out_hbm.shape[2]
    nw = 32
    tpt = T // nw
    nch = 4
    ct = tpt // nch
    wid = lax.axis_index("s") * 2 + lax.axis_index("c")
    base = wid * tpt
    pltpu.sync_copy(pos_hbm.at[0, pl.ds(base, tpt)], p0_v)
    pltpu.sync_copy(pos_hbm.at[1, pl.ds(base, tpt)], p1_v)
    pltpu.sync_copy(rw_hbm.at[0, pl.ds(base, tpt)], w_v.at[0, pl.ds(0, tpt)])
    pltpu.sync_copy(rw_hbm.at[1, pl.ds(base, tpt)], w_v.at[1, pl.ds(0, tpt)])

    def g0(c):
        b = c % 2
        return pltpu.make_async_copy(
            y_hbm.at[p0_v.at[pl.ds(c * ct, ct)]], bufs[2 * b], gsems[2 * b])

    def g1(c):
        b = c % 2
        return pltpu.make_async_copy(
            y_hbm.at[p1_v.at[pl.ds(c * ct, ct)]], bufs[2 * b + 1],
            gsems[2 * b + 1])

    def wr(c):
        b = c % 2
        return pltpu.make_async_copy(
            bufs[2 * b], out_hbm.at[0, pl.ds(base + c * ct, ct)], wsems[b])

    g0(0).start()
    g1(0).start()
    g0(1).start()
    g1(1).start()
    for c in range(nch):
        g0(c).wait()
        g1(c).wait()
        b = c % 2
        buf0, buf1 = bufs[2 * b], bufs[2 * b + 1]

        def addrow(r, carry):
            tok = c * ct + r
            s0 = w_v[0, pl.ds(tok, 16)][0]
            s1 = w_v[1, pl.ds(tok, 16)][0]
            for cc in range(D // 16):
                col = cc * 16
                buf0[r, pl.ds(col, 16)] = (buf0[r, pl.ds(col, 16)] * s0
                                           + buf1[r, pl.ds(col, 16)] * s1)
            return carry
        lax.fori_loop(0, ct, addrow, 0)
        wr(c).start()
        if c + 2 < nch:
            wr(c).wait()
            g0(c + 2).start()
            g1(c + 2).start()
    wr(nch - 2).wait()
    wr(nch - 1).wait()


def _combine(y, pos, rw, T, D):
    mesh = plsc.VectorSubcoreMesh(core_axis_name="c", subcore_axis_name="s")
    ct = T // 32 // 4
    return pl.kernel(
        _combine_body,
        out_type=jax.ShapeDtypeStruct((1, T, D), jnp.float32),
        mesh=mesh,
        compiler_params=pltpu.CompilerParams(needs_layout_passes=False),
        scratch_types=[
            pltpu.VMEM((T // 32,), jnp.int32),
            pltpu.VMEM((T // 32,), jnp.int32),
            pltpu.VMEM((2, T // 32 + 16), jnp.float32),
            [pltpu.VMEM((ct, D), jnp.float32) for _ in range(4)],
            [pltpu.SemaphoreType.DMA for _ in range(4)],
            [pltpu.SemaphoreType.DMA for _ in range(2)],
        ],
    )(y, pos, rw)


def kernel(hidden_states, gate_w, w1, w3, w2):
    orig_shape = hidden_states.shape
    D = orig_shape[-1]
    x = hidden_states.reshape(-1, D)
    T = x.shape[0]
    pos, rw, ex, used = _router(x, gate_w)
    xs_a, gidx = _dispatch_a(x, pos, used)
    xs_b = _dispatch_b(x, gidx, used)
    exf = ex.reshape(EX_W)
    y = _ffn_half(exf, xs_a, w1, w3, w2, 0)
    y = _ffn_half(exf, xs_b, w1, w3, w2, SLOTS_A // TILE_M, y_prev=y)
    out = _combine(y, pos, rw, T, D)
    return out.reshape(orig_shape)


# speedup vs baseline: 1.2763x; 1.0017x over previous
import functools

import jax
import jax.numpy as jnp
from jax import lax
from jax.experimental import pallas as pl
from jax.experimental.pallas import tpu as pltpu
from jax.experimental.pallas import tpu_sc as plsc

TOPK = 2
TILE_M = 256
TILE_SHIFT = 8
NUM_TILES = 24
NSLOT = NUM_TILES * TILE_M
EX_W = 32
SCAN_CHUNK = 512


def _router_body(x_ref, gw_ref, pos_ref, rw_ref, ex_ref, used_ref):
    x = x_ref[...]
    gw = gw_ref[...]
    E = gw.shape[0]
    T = x.shape[0]
    logits = lax.dot_general(gw, x, (((1,), (1,)), ((), ())),
                             preferred_element_type=jnp.float32)
    row = lax.broadcasted_iota(jnp.int32, (E, T), 0)
    v0 = jnp.max(logits, axis=0, keepdims=True)
    a0 = jnp.min(jnp.where(logits == v0, row, E), axis=0, keepdims=True)
    masked = jnp.where(row == a0, -jnp.inf, logits)
    v1 = jnp.max(masked, axis=0, keepdims=True)
    a1 = jnp.min(jnp.where(masked == v1, row, E), axis=0, keepdims=True)
    d = jnp.exp(v1 - v0)
    w0 = 1.0 / (1.0 + d)
    w1 = d / (1.0 + d)

    oh0 = (row == a0).astype(jnp.float32)
    oh1 = (row == a1).astype(jnp.float32)

    C = SCAN_CHUNK
    ci = lax.broadcasted_iota(jnp.int32, (C, C), 0)
    cj = lax.broadcasted_iota(jnp.int32, (C, C), 1)
    upper_incl = (ci <= cj).astype(jnp.float32)
    carry = jnp.zeros((E, 1), jnp.float32)
    ranks = []
    for oh in (oh0, oh1):
        chunks = []
        for c in range(T // C):
            ohc = oh[:, c * C:(c + 1) * C]
            run = lax.dot_general(ohc, upper_incl, (((1,), (0,)), ((), ())),
                                  preferred_element_type=jnp.float32) + carry
            chunks.append(jnp.sum(run * ohc, axis=0, keepdims=True))
            carry = run[:, C - 1:C]
        ranks.append(jnp.concatenate(chunks, axis=1) - 1.0)
    counts = carry

    counts_i = counts.astype(jnp.int32)
    tiles = lax.shift_right_logical(counts_i + (TILE_M - 1), TILE_SHIFT)
    tiles_f = tiles.astype(jnp.float32)
    ei = lax.broadcasted_iota(jnp.int32, (E, E), 0)
    ej = lax.broadcasted_iota(jnp.int32, (E, E), 1)
    strict_lower = (ej < ei).astype(jnp.float32)
    tbase = lax.dot_general(strict_lower, tiles_f, (((1,), (0,)), ((), ())),
                            preferred_element_type=jnp.float32)
    pbase = tbase * float(TILE_M)

    pos0 = jnp.sum(oh0 * pbase, axis=0, keepdims=True) + ranks[0]
    pos1 = jnp.sum(oh1 * pbase, axis=0, keepdims=True) + ranks[1]
    pos_ref[0:1, :] = pos0.astype(jnp.int32)
    pos_ref[1:2, :] = pos1.astype(jnp.int32)
    rw_ref[0:1, :] = w0
    rw_ref[1:2, :] = w1

    g = lax.broadcasted_iota(jnp.int32, (1, EX_W), 1)
    owner = jnp.sum((tbase <= g.astype(jnp.float32)).astype(jnp.float32),
                    axis=0, keepdims=True).astype(jnp.int32) - 1
    total = jnp.sum(tiles_f).astype(jnp.int32)
    ex_ref[...] = jnp.where(g < total, owner, -1)
    used_ref[...] = jnp.zeros((1, 16), jnp.int32) + total * TILE_M


def _router(x, gate_w):
    T, D = x.shape
    return pl.pallas_call(
        _router_body,
        out_shape=(
            jax.ShapeDtypeStruct((2, T), jnp.int32),
            jax.ShapeDtypeStruct((2, T), jnp.float32),
            jax.ShapeDtypeStruct((1, EX_W), jnp.int32),
            jax.ShapeDtypeStruct((1, 16), jnp.int32),
        ),
    )(x, gate_w)


SLOTS_A = 4 * TILE_M
SLOTS_B = NSLOT - SLOTS_A


def _pipelined_gather(x_hbm, dst_hbm, idx_ref, dst_base, idx_base, used,
                      glob_base, ch, nch, rows, gsems, wsems):
    def copy_in(c):
        b = c % 2
        idx = idx_ref.at[pl.ds(idx_base + c * ch, ch)]
        return pltpu.make_async_copy(x_hbm.at[idx], rows[b], gsems[b])

    def copy_out(c):
        b = c % 2
        return pltpu.make_async_copy(
            rows[b], dst_hbm.at[pl.ds(dst_base + c * ch, ch)], wsems[b])

    live = [glob_base + c * ch < used for c in range(nch)]
    pl.when(live[0])(lambda: copy_in(0).start())
    if nch > 1:
        pl.when(live[1])(lambda: copy_in(1).start())
    for c in range(nch):
        def drain(c=c):
            copy_in(c).wait()
            copy_out(c).start()
        pl.when(live[c])(drain)
        if c + 2 < nch:
            pl.when(live[c])(lambda c=c: copy_out(c).wait())
            pl.when(live[c + 2])(lambda c=c: copy_in(c + 2).start())
    for c in range(max(0, nch - 2), nch):
        pl.when(live[c])(lambda c=c: copy_out(c).wait())


def _dispatch_a_body(x_hbm, pos_hbm, used_hbm,
                     xs_hbm, gidx_hbm,
                     pos_v, used_v, gidx_v, rows0, rows1,
                     sg0, sg1, sw0, sw1):
    T, D = x_hbm.shape
    nw = 32
    spt = NSLOT // nw
    spa = SLOTS_A // nw
    ch = spa // 2
    wid = lax.axis_index("s") * 2 + lax.axis_index("c")
    base = wid * spt
    basea = wid * spa

    with jax.named_scope("disp_meta"):
        pltpu.sync_copy(pos_hbm, pos_v)
        pltpu.sync_copy(used_hbm, used_v)

        def init(i, c):
            off = base + i * 16
            gidx_v[pl.ds(off, 16)] = jnp.bitwise_and(
                lax.iota(jnp.int32, 16) + off, T - 1)
            offa = basea + i * 16
            gidx_v[pl.ds(offa, 16)] = jnp.bitwise_and(
                lax.iota(jnp.int32, 16) + offa, T - 1)
            return c
        lax.fori_loop(0, spt // 16, init, 0)

        def scat(j, c):
            tid = lax.iota(jnp.int32, 16) + j * 16
            for k in range(TOPK):
                p = pos_v[k, pl.ds(j * 16, 16)]
                plsc.store_scatter(gidx_v, [p], tid)
            return c
        lax.fori_loop(0, T // 16, scat, 0)

        pltpu.sync_copy(gidx_v.at[pl.ds(base, spt)], gidx_hbm.at[pl.ds(base, spt)])

    used = used_v[0, pl.ds(0, 16)][0]
    with jax.named_scope("disp_gather"):
        _pipelined_gather(x_hbm, xs_hbm, gidx_v, basea, basea, used,
                          basea, ch, 2, (rows0, rows1), (sg0, sg1), (sw0, sw1))


def _dispatch_a(x, pos, used):
    T, D = x.shape
    mesh = plsc.VectorSubcoreMesh(core_axis_name="c", subcore_axis_name="s")
    ch = SLOTS_A // 32 // 2
    return pl.kernel(
        _dispatch_a_body,
        out_type=(
            jax.ShapeDtypeStruct((SLOTS_A, D), jnp.float32),
            jax.ShapeDtypeStruct((NSLOT,), jnp.int32),
        ),
        mesh=mesh,
        compiler_params=pltpu.CompilerParams(needs_layout_passes=False),
        scratch_types=[
            pltpu.VMEM((2, T), jnp.int32),
            pltpu.VMEM((1, 16), jnp.int32),
            pltpu.VMEM((NSLOT,), jnp.int32),
            pltpu.VMEM((ch, D), jnp.float32),
            pltpu.VMEM((ch, D), jnp.float32),
            pltpu.SemaphoreType.DMA,
            pltpu.SemaphoreType.DMA,
            pltpu.SemaphoreType.DMA,
            pltpu.SemaphoreType.DMA,
        ],
    )(x, pos, used)


def _dispatch_b_body(x_hbm, gidx_hbm, used_hbm, xs_hbm,
                     gixb_v, used_v, rows0, rows1, sg0, sg1, sw0, sw1):
    T, D = x_hbm.shape
    nw = 32
    spb = SLOTS_B // nw
    ch = spb // 4
    wid = lax.axis_index("s") * 2 + lax.axis_index("c")
    base = wid * spb
    gbase = SLOTS_A + base

    pltpu.sync_copy(gidx_hbm.at[pl.ds(gbase, spb)], gixb_v)
    pltpu.sync_copy(used_hbm, used_v)
    used = used_v[0, pl.ds(0, 16)][0]
    _pipelined_gather(x_hbm, xs_hbm, gixb_v, base, 0, used,
                      gbase, ch, 4, (rows0, rows1), (sg0, sg1), (sw0, sw1))


def _dispatch_b(x, gidx, used):
    T, D = x.shape
    mesh = plsc.VectorSubcoreMesh(core_axis_name="c", subcore_axis_name="s")
    ch = SLOTS_B // 32 // 4
    return pl.kernel(
        _dispatch_b_body,
        out_type=jax.ShapeDtypeStruct((SLOTS_B, D), jnp.float32),
        mesh=mesh,
        compiler_params=pltpu.CompilerParams(needs_layout_passes=False),
        scratch_types=[
            pltpu.VMEM((SLOTS_B // 32,), jnp.int32),
            pltpu.VMEM((1, 16), jnp.int32),
            pltpu.VMEM((ch, D), jnp.float32),
            pltpu.VMEM((ch, D), jnp.float32),
            pltpu.SemaphoreType.DMA,
            pltpu.SemaphoreType.DMA,
            pltpu.SemaphoreType.DMA,
            pltpu.SemaphoreType.DMA,
        ],
    )(x, gidx, used)


def _ffn_body(ex_ref, x_ref, w1_ref, w3_ref, w2_ref, y_ref, lo=0):
    @pl.when(ex_ref[pl.program_id(0) + lo] >= 0)
    def _():
        x = x_ref[...]
        g = lax.dot_general(x, w1_ref[0], (((1,), (1,)), ((), ())),
                            preferred_element_type=jnp.float32)
        u = lax.dot_general(x, w3_ref[0], (((1,), (1,)), ((), ())),
                            preferred_element_type=jnp.float32)
        h = g * jax.nn.sigmoid(g) * u
        y_ref[...] = lax.dot_general(h, w2_ref[0], (((1,), (1,)), ((), ())),
                                     preferred_element_type=jnp.float32)


def _ffn_half(ex, xs_half, w1, w3, w2, lo, y_prev=None):
    E, F, D = w1.shape
    nt = xs_half.shape[0] // TILE_M

    def wsel(g, ex_s):
        return (jnp.maximum(ex_s[g + lo], 0), 0, 0)

    in_specs = [
        pl.BlockSpec((TILE_M, D), lambda g, ex_s: (g, 0)),
        pl.BlockSpec((1, F, D), wsel),
        pl.BlockSpec((1, F, D), wsel),
        pl.BlockSpec((1, D, F), wsel),
    ]
    args = [ex, xs_half, w1, w3, w2]
    kwargs = {}
    if y_prev is not None:
        in_specs.append(pl.BlockSpec(memory_space=pl.ANY))
        args.append(y_prev)
        kwargs["input_output_aliases"] = {5: 0}

    def body(ex_ref, x_ref, w1_ref, w3_ref, w2_ref, *rest):
        y_ref = rest[-1]
        _ffn_body(ex_ref, x_ref, w1_ref, w3_ref, w2_ref, y_ref, lo=lo)

    grid_spec = pltpu.PrefetchScalarGridSpec(
        num_scalar_prefetch=1,
        grid=(nt,),
        in_specs=in_specs,
        out_specs=pl.BlockSpec((TILE_M, D), lambda g, ex_s: (g + lo, 0)),
    )
    return pl.pallas_call(
        body,
        grid_spec=grid_spec,
        out_shape=jax.ShapeDtypeStruct((NSLOT, D), jnp.float32),
        **kwargs,
    )(*args)


def _combine_body(y_hbm, pos_hbm, rw_hbm, out_hbm, p0_v, p1_v, w_v,
                  bufs, gsems, wsems):
    T = out_hbm.shape[1]
    D = out_hbm.shape[2]
    nw = 32
    tpt = T // nw
    nch = 4
    ct = tpt // nch
    wid = lax.axis_index("s") * 2 + lax.axis_index("c")
    base = wid * tpt
    pltpu.sync_copy(pos_hbm.at[0, pl.ds(base, tpt)], p0_v)
    pltpu.sync_copy(pos_hbm.at[1, pl.ds(base, tpt)], p1_v)
    pltpu.sync_copy(rw_hbm.at[0, pl.ds(base, tpt)], w_v.at[0, pl.ds(0, tpt)])
    pltpu.sync_copy(rw_hbm.at[1, pl.ds(base, tpt)], w_v.at[1, pl.ds(0, tpt)])

    def g0(c):
        b = c % 3
        return pltpu.make_async_copy(
            y_hbm.at[p0_v.at[pl.ds(c * ct, ct)]], bufs[2 * b], gsems[2 * b])

    def g1(c):
        b = c % 3
        return pltpu.make_async_copy(
            y_hbm.at[p1_v.at[pl.ds(c * ct, ct)]], bufs[2 * b + 1],
            gsems[2 * b + 1])

    def wr(c):
        b = c % 3
        return pltpu.make_async_copy(
            bufs[2 * b], out_hbm.at[0, pl.ds(base + c * ct, ct)], wsems[b])

    g0(0).start()
    g1(0).start()
    g0(1).start()
    g1(1).start()
    g0(2).start()
    g1(2).start()
    for c in range(nch):
        g0(c).wait()
        g1(c).wait()
        b = c % 3
        buf0, buf1 = bufs[2 * b], bufs[2 * b + 1]

        def addrow(r, carry):
            tok = c * ct + r
            s0 = w_v[0, pl.ds(tok, 16)][0]
            s1 = w_v[1, pl.ds(tok, 16)][0]
            for cc in range(D // 16):
                col = cc * 16
                buf0[r, pl.ds(col, 16)] = (buf0[r, pl.ds(col, 16)] * s0
                                           + buf1[r, pl.ds(col, 16)] * s1)
            return carry
        lax.fori_loop(0, ct, addrow, 0)
        wr(c).start()
        if c + 3 < nch:
            wr(c).wait()
            g0(c + 3).start()
            g1(c + 3).start()
    for c in range(max(0, nch - 3), nch):
        wr(c).wait()


def _combine(y, pos, rw, T, D):
    mesh = plsc.VectorSubcoreMesh(core_axis_name="c", subcore_axis_name="s")
    ct = T // 32 // 4
    return pl.kernel(
        _combine_body,
        out_type=jax.ShapeDtypeStruct((1, T, D), jnp.float32),
        mesh=mesh,
        compiler_params=pltpu.CompilerParams(needs_layout_passes=False),
        scratch_types=[
            pltpu.VMEM((T // 32,), jnp.int32),
            pltpu.VMEM((T // 32,), jnp.int32),
            pltpu.VMEM((2, T // 32 + 16), jnp.float32),
            [pltpu.VMEM((ct, D), jnp.float32) for _ in range(6)],
            [pltpu.SemaphoreType.DMA for _ in range(6)],
            [pltpu.SemaphoreType.DMA for _ in range(3)],
        ],
    )(y, pos, rw)


def kernel(hidden_states, gate_w, w1, w3, w2):
    orig_shape = hidden_states.shape
    D = orig_shape[-1]
    x = hidden_states.reshape(-1, D)
    T = x.shape[0]
    pos, rw, ex, used = _router(x, gate_w)
    xs_a, gidx = _dispatch_a(x, pos, used)
    xs_b = _dispatch_b(x, gidx, used)
    exf = ex.reshape(EX_W)
    y = _ffn_half(exf, xs_a, w1, w3, w2, 0)
    y = _ffn_half(exf, xs_b, w1, w3, w2, SLOTS_A // TILE_M, y_prev=y)
    out = _combine(y, pos, rw, T, D)
    return out.reshape(orig_shape)
